# baseline (speedup 1.0000x reference)
import functools

import jax
import jax.numpy as jnp
from jax import lax
from jax.experimental import pallas as pl
from jax.experimental.pallas import tpu as pltpu
from jax.experimental.pallas import tpu_sc as plsc

VOCAB = 100000
DIM = 128
BATCH = 1024


def _gather_sc(table, idx):
    info = plsc.get_sparse_core_info()
    nw = info.num_cores * info.num_subcores
    bpw = BATCH // nw
    mesh = plsc.VectorSubcoreMesh(core_axis_name="c", subcore_axis_name="s")

    @functools.partial(
        pl.kernel,
        mesh=mesh,
        out_type=jax.ShapeDtypeStruct((BATCH, DIM), jnp.float32),
        scratch_types=[
            pltpu.VMEM((bpw,), jnp.int32),
            pltpu.VMEM((bpw, DIM), jnp.float32),
            pltpu.SemaphoreType.DMA,
        ],
    )
    def gather_kernel(table_hbm, idx_hbm, out_hbm, idx_v, rows_v, sem):
        wid = lax.axis_index("s") * info.num_cores + lax.axis_index("c")
        base = wid * bpw
        pltpu.sync_copy(idx_hbm.at[pl.ds(base, bpw)], idx_v)
        pltpu.async_copy(table_hbm.at[idx_v], rows_v, sem).wait()
        pltpu.sync_copy(rows_v, out_hbm.at[pl.ds(base, bpw)])

    return gather_kernel(table, idx)


_VT = 2048
_NT = (VOCAB + _VT - 1) // _VT
_NFULL = VOCAB // _VT
_NB = 4


_TA = 1664
_TB = 128


def _proj_kernel(e_ref, w_ref, b_ref, o_hbm, *scratch):
    bufs = scratch[:_NB]
    sems = scratch[_NB : 2 * _NB]
    buf_t, sem_a, sem_b = scratch[2 * _NB :]
    j = pl.program_id(0)
    acc = (
        jnp.dot(e_ref[...], w_ref[...], preferred_element_type=jnp.float32)
        + b_ref[...]
    )

    @pl.when(j < _NFULL)
    def _():
        for s in range(_NB):

            @pl.when(jax.lax.rem(j, _NB) == s)
            def _(s=s):
                @pl.when(j >= _NB)
                def _():
                    pltpu.make_async_copy(
                        bufs[s], o_hbm.at[:, pl.ds((j - _NB) * _VT, _VT)], sems[s]
                    ).wait()

                bufs[s][...] = acc
                pltpu.make_async_copy(
                    bufs[s], o_hbm.at[:, pl.ds(j * _VT, _VT)], sems[s]
                ).start()

    @pl.when(j == _NT - 1)
    def _():
        buf_t[...] = acc
        pltpu.make_async_copy(
            buf_t.at[:, pl.ds(0, _TA)], o_hbm.at[:, pl.ds(j * _VT, _TA)], sem_a
        ).start()
        start = pl.multiple_of(j * _VT + _TA, 128)
        pltpu.make_async_copy(
            buf_t.at[:, pl.ds(_TA, _TB)], o_hbm.at[:, pl.ds(start, _TB)], sem_b
        ).start()
        for s in range(_NB):
            jl = _NFULL - 1 - ((_NFULL - 1 - s) % _NB)
            pltpu.make_async_copy(
                bufs[s], o_hbm.at[:, pl.ds(jl * _VT, _VT)], sems[s]
            ).wait()
        pltpu.make_async_copy(
            buf_t.at[:, pl.ds(0, _TA)], o_hbm.at[:, pl.ds(j * _VT, _TA)], sem_a
        ).wait()
        pltpu.make_async_copy(
            buf_t.at[:, pl.ds(_TA, _TB)], o_hbm.at[:, pl.ds(start, _TB)], sem_b
        ).wait()


def _project(e, W, b):
    b2 = b.reshape(1, VOCAB)
    return pl.pallas_call(
        _proj_kernel,
        grid=(_NT,),
        in_specs=[
            pl.BlockSpec((BATCH, DIM), lambda j: (0, 0)),
            pl.BlockSpec((DIM, _VT), lambda j: (0, j)),
            pl.BlockSpec((1, _VT), lambda j: (0, j)),
        ],
        out_specs=pl.BlockSpec(memory_space=pl.ANY),
        out_shape=jax.ShapeDtypeStruct((BATCH, VOCAB), jnp.float32),
        scratch_shapes=(
            [pltpu.VMEM((BATCH, _VT), jnp.float32) for _ in range(_NB)]
            + [pltpu.SemaphoreType.DMA for _ in range(_NB)]
            + [
                pltpu.VMEM((BATCH, _VT), jnp.float32),
                pltpu.SemaphoreType.DMA,
                pltpu.SemaphoreType.DMA,
            ]
        ),
    )(e, W, b2)


_GR = 16


def _gather_tc(table, idx):
    def body(idx_ref, *refs):
        i = pl.program_id(0)
        t_refs = refs[:_GR]
        o_ref = refs[_GR]
        sub = jax.lax.broadcasted_iota(jnp.int32, (8, DIM), 0)
        for k in range(_GR):
            rem = jax.lax.rem(idx_ref[_GR * i + k], 8)
            rows8 = t_refs[k][...]
            row = jnp.sum(
                jnp.where(sub == rem, rows8, 0.0), axis=0, keepdims=True
            )
            o_ref[pl.ds(k, 1), :] = row

    def t_spec(k):
        return pl.BlockSpec(
            (8, DIM), lambda i, idx_ref, k=k: (idx_ref[_GR * i + k] // 8, 0)
        )

    grid_spec = pltpu.PrefetchScalarGridSpec(
        num_scalar_prefetch=1,
        grid=(BATCH // _GR,),
        in_specs=[t_spec(k) for k in range(_GR)],
        out_specs=pl.BlockSpec((_GR, DIM), lambda i, idx_ref: (i, 0)),
    )
    return pl.pallas_call(
        body,
        grid_spec=grid_spec,
        out_shape=jax.ShapeDtypeStruct((BATCH, DIM), jnp.float32),
    )(idx, *([table] * _GR))


def kernel(x, table, W, b):
    idx = x.astype(jnp.int32)
    e = _gather_tc(table, idx)
    return _project(e, W, b)


# speedup vs baseline: 3.5320x; 3.5320x over previous
import functools

import jax
import jax.numpy as jnp
from jax import lax
from jax.experimental import pallas as pl
from jax.experimental.pallas import tpu as pltpu
from jax.experimental.pallas import tpu_sc as plsc

VOCAB = 100000
DIM = 128
BATCH = 1024


def _gather_sc(table, idx):
    info = plsc.get_sparse_core_info()
    nw = info.num_cores * info.num_subcores
    bpw = BATCH // nw
    mesh = plsc.VectorSubcoreMesh(core_axis_name="c", subcore_axis_name="s")

    @functools.partial(
        pl.kernel,
        mesh=mesh,
        out_type=jax.ShapeDtypeStruct((BATCH, DIM), jnp.float32),
        scratch_types=[
            pltpu.VMEM((bpw,), jnp.int32),
            pltpu.VMEM((bpw, DIM), jnp.float32),
            pltpu.SemaphoreType.DMA,
        ],
    )
    def gather_kernel(table_hbm, idx_hbm, out_hbm, idx_v, rows_v, sem):
        wid = lax.axis_index("s") * info.num_cores + lax.axis_index("c")
        base = wid * bpw
        pltpu.sync_copy(idx_hbm.at[pl.ds(base, bpw)], idx_v)
        pltpu.async_copy(table_hbm.at[idx_v], rows_v, sem).wait()
        pltpu.sync_copy(rows_v, out_hbm.at[pl.ds(base, bpw)])

    return gather_kernel(table, idx)


_VT = 2048
_NT = (VOCAB + _VT - 1) // _VT
_NFULL = VOCAB // _VT
_TAIL = VOCAB - _NFULL * _VT
_NB = 4


def _proj_kernel(wt_ref, et_ref, b_ref, o_hbm, *scratch):
    bufs = scratch[:_NB]
    sems = scratch[_NB : 2 * _NB]
    buf_t, sem_t = scratch[2 * _NB :]
    j = pl.program_id(0)
    acc = jnp.dot(wt_ref[...], et_ref[...], preferred_element_type=jnp.float32)
    acc = acc + b_ref[...][0, :, None]

    @pl.when(j < _NFULL)
    def _():
        for s in range(_NB):

            @pl.when(lax.rem(j, _NB) == s)
            def _(s=s):
                @pl.when(j >= _NB)
                def _():
                    pltpu.make_async_copy(
                        bufs[s], o_hbm.at[pl.ds((j - _NB) * _VT, _VT), :], sems[s]
                    ).wait()

                bufs[s][...] = acc
                pltpu.make_async_copy(
                    bufs[s], o_hbm.at[pl.ds(j * _VT, _VT), :], sems[s]
                ).start()

    @pl.when(j == _NT - 1)
    def _():
        buf_t[...] = acc
        pltpu.make_async_copy(
            buf_t.at[pl.ds(0, _TAIL), :],
            o_hbm.at[pl.ds(_NFULL * _VT, _TAIL), :],
            sem_t,
        ).start()
        for s in range(_NB):
            jl = _NFULL - 1 - ((_NFULL - 1 - s) % _NB)
            pltpu.make_async_copy(
                bufs[s], o_hbm.at[pl.ds(jl * _VT, _VT), :], sems[s]
            ).wait()
        pltpu.make_async_copy(
            buf_t.at[pl.ds(0, _TAIL), :],
            o_hbm.at[pl.ds(_NFULL * _VT, _TAIL), :],
            sem_t,
        ).wait()


def _project_t(Wt, eT, b):
    b2 = b.reshape(1, VOCAB)
    return pl.pallas_call(
        _proj_kernel,
        grid=(_NT,),
        in_specs=[
            pl.BlockSpec((_VT, DIM), lambda j: (j, 0)),
            pl.BlockSpec((DIM, BATCH), lambda j: (0, 0)),
            pl.BlockSpec((1, _VT), lambda j: (0, j)),
        ],
        out_specs=pl.BlockSpec(memory_space=pl.ANY),
        out_shape=jax.ShapeDtypeStruct((VOCAB, BATCH), jnp.float32),
        scratch_shapes=(
            [pltpu.VMEM((_VT, BATCH), jnp.float32) for _ in range(_NB)]
            + [pltpu.SemaphoreType.DMA for _ in range(_NB)]
            + [pltpu.VMEM((_VT, BATCH), jnp.float32), pltpu.SemaphoreType.DMA]
        ),
    )(Wt, eT, b2)


def kernel(x, table, W, b):
    idx = x.astype(jnp.int32)
    e = _gather_sc(table, idx)
    out_t = _project_t(W.T, e.T, b)
    return out_t.T
